# baseline (speedup 1.0000x reference)
import functools

import jax
import jax.numpy as jnp
from jax import lax
from jax.experimental import pallas as pl
from jax.experimental.pallas import tpu as pltpu
from jax.experimental.pallas import tpu_sc as plsc

DE = 5
BS = 8
N_NODES = 256
N_SLOTS = N_NODES * N_NODES
S_TOTAL = BS * N_SLOTS
NW = 32
PER_W = S_TOTAL // NW
CHUNK = 2048
NCHUNK = PER_W // CHUNK
GROUPS = CHUNK // 16
CH5 = CHUNK * DE


def _sc_body(xf, pf, gf, qtab, probf, samp, xb, pb, gb, qb, ob, sb):
    cid = lax.axis_index("c")
    sid = lax.axis_index("s")
    wid = cid * 16 + sid
    batch = wid // (NW // BS)
    pltpu.sync_copy(qtab.at[batch], qb)

    iota = lax.iota(jnp.int32, 16)
    idx5 = iota * 5

    @pl.loop(0, NCHUNK)
    def _chunk(t):
        base = wid * PER_W + t * CHUNK
        pltpu.sync_copy(xf.at[pl.ds(base * DE, CH5)], xb)
        pltpu.sync_copy(pf.at[pl.ds(base * DE, CH5)], pb)
        pltpu.sync_copy(gf.at[pl.ds(base * DE, CH5)], gb)

        @pl.loop(0, GROUPS, unroll=4)
        def _group(g):
            goff = g * (16 * DE)
            x = [plsc.load_gather(xb, [idx5 + (goff + c)]) for c in range(DE)]
            p = [plsc.load_gather(pb, [idx5 + (goff + c)]) for c in range(DE)]
            eg = [plsc.load_gather(gb, [idx5 + (goff + c)]) for c in range(DE)]

            def rbf16(v):
                b = plsc.bitcast(v, jnp.int32)
                b = (b + 0x7FFF + ((b >> 16) & 1)) & ~0xFFFF
                return plsc.bitcast(b, jnp.float32)

            x = [rbf16(x[c]) for c in range(DE)]

            m = p[0]
            for c in range(1, DE):
                m = jnp.maximum(m, p[c])
            e = [jnp.exp(p[c] - m) for c in range(DE)]

            w = []
            for j in range(DE):
                acc = x[0] * qb[50 + j * DE]
                for c in range(1, DE):
                    acc = acc + x[c] * qb[50 + j * DE + c]
                acc = jnp.where(acc == 0.0, 1e-6, acc)
                w.append(e[j] / acc)

            u = []
            den = None
            for k in range(DE):
                left = x[0] * qb[k * DE]
                for c in range(1, DE):
                    left = left + x[c] * qb[k * DE + c]
                s = w[0] * qb[25 + k]
                for j in range(1, DE):
                    s = s + w[j] * qb[25 + j * DE + k]
                uk = left * s
                u.append(uk)
                den = uk if den is None else den + uk
            den = jnp.where(den == 0.0, 1e-5, den)

            prob = [u[k] / den for k in range(DE)]

            best = (prob[0] + 1e-30) * jnp.exp(eg[0])
            bidx = jnp.zeros((16,), jnp.int32)
            for k in range(1, DE):
                val = (prob[k] + 1e-30) * jnp.exp(eg[k])
                gt = val > best
                best = jnp.where(gt, val, best)
                bidx = jnp.where(gt, k, bidx)

            for c in range(DE):
                plsc.store_scatter(ob, [idx5 + (goff + c)], prob[c])
            sb[pl.ds(g * 16, 16)] = bidx

        pltpu.sync_copy(ob, probf.at[pl.ds(base * DE, CH5)])
        pltpu.sync_copy(sb, samp.at[pl.ds(base, CHUNK)])


@jax.jit
def _sc_main(xf, pf, gf, qtab):
    mesh = plsc.VectorSubcoreMesh(core_axis_name="c", subcore_axis_name="s")
    f = pl.kernel(
        _sc_body,
        out_type=[
            jax.ShapeDtypeStruct((S_TOTAL * DE,), jnp.float32),
            jax.ShapeDtypeStruct((S_TOTAL,), jnp.int32),
        ],
        mesh=mesh,
        compiler_params=pltpu.CompilerParams(
            use_tc_tiling_on_sc=False, needs_layout_passes=False
        ),
        scratch_types=[
            pltpu.VMEM((CH5,), jnp.float32),
            pltpu.VMEM((CH5,), jnp.float32),
            pltpu.VMEM((CH5,), jnp.float32),
            pltpu.VMEM((80, 16), jnp.float32),
            pltpu.VMEM((CH5,), jnp.float32),
            pltpu.VMEM((CHUNK,), jnp.int32),
        ],
    )
    return f(xf, pf, gf, qtab)


def _sym_body(raw_ref, out_ref):
    r = raw_ref[0].astype(jnp.float32)
    row = lax.broadcasted_iota(jnp.int32, (N_NODES, N_NODES), 0)
    col = lax.broadcasted_iota(jnp.int32, (N_NODES, N_NODES), 1)
    up = jnp.where(col > row, r, 0.0)
    out_ref[0] = (up + up.T).astype(jnp.int32)


@jax.jit
def _tc_symmetrize(raw):
    return pl.pallas_call(
        _sym_body,
        grid=(BS,),
        in_specs=[pl.BlockSpec((1, N_NODES, N_NODES), lambda b: (b, 0, 0))],
        out_specs=pl.BlockSpec((1, N_NODES, N_NODES), lambda b: (b, 0, 0)),
        out_shape=jax.ShapeDtypeStruct((BS, N_NODES, N_NODES), jnp.int32),
    )(raw)


def kernel(X_t, pred_E, Qt, Qsb, Qtb):
    bs, n = X_t.shape[0], X_t.shape[1]
    de = X_t.shape[-1]
    g = jax.random.gumbel(jax.random.key(42), (bs, n * n, de), jnp.float32)

    def _round_bf16(a):
        b = lax.bitcast_convert_type(a, jnp.int32)
        b = (b + 0x7FFF + ((b >> 16) & 1)) & ~0xFFFF
        return lax.bitcast_convert_type(b, jnp.float32)

    qt_r = _round_bf16(Qt)
    qtb_r = _round_bf16(Qtb)
    qtab = jnp.concatenate(
        [qt_r.reshape(bs, de * de), Qsb.reshape(bs, de * de), qtb_r.reshape(bs, de * de)],
        axis=1,
    )
    qtab = jnp.pad(qtab, ((0, 0), (0, 80 - 3 * de * de)))
    qtab = jnp.broadcast_to(qtab[:, :, None], (bs, 80, 16))

    probf, samp = _sc_main(
        X_t.reshape(-1), pred_E.reshape(-1), g.reshape(-1), qtab
    )
    prob = probf.reshape(bs, n * n, de)
    E_t = _tc_symmetrize(samp.reshape(bs, n, n))
    return prob, E_t
